# baseline (speedup 1.0000x reference)
import functools

import jax
import jax.numpy as jnp
from jax import lax
from jax.experimental import pallas as pl
from jax.experimental.pallas import tpu as pltpu
from jax.experimental.pallas import tpu_sc as plsc

NUM_ROWS = 100000
DIM = 64
XROWS = 4096
XCOLS = 50
NC = 2
NS = 16
NW = NC * NS
RPW = XROWS // NW
G = 4
NG = RPW // G

_mesh = plsc.VectorSubcoreMesh(core_axis_name="c", subcore_axis_name="s")


@functools.partial(
    pl.kernel,
    mesh=_mesh,
    compiler_params=pltpu.CompilerParams(use_tc_tiling_on_sc=False),
    out_type=jax.ShapeDtypeStruct((XROWS, 56, 128), jnp.float32),
    scratch_types=(
        [pltpu.VMEM((RPW, XCOLS), jnp.int32)]
        + [pltpu.VMEM((2, G, XCOLS, DIM), jnp.float32)]
        + [pltpu.SemaphoreType.DMA] * 4
    ),
)
def _embed_gather(x_hbm, table_hbm, out_hbm, idx_v, rows_v, g0, g1, o0, o1):
    gsem = (g0, g1)
    osem = (o0, o1)
    wid = lax.axis_index("s") * NC + lax.axis_index("c")
    base = wid * RPW
    pltpu.sync_copy(x_hbm.at[pl.ds(base, RPW)], idx_v)

    def fire_group(g, p):
        for k in range(G):
            pltpu.make_async_copy(
                table_hbm.at[idx_v.at[g * G + k]], rows_v.at[p].at[k],
                gsem[p]).start()

    def wait_group(p):
        for k in range(G):
            pltpu.make_async_copy(
                table_hbm.at[idx_v.at[0]], rows_v.at[p].at[k],
                gsem[p]).wait()

    def write(g, p):
        return pltpu.make_async_copy(
            rows_v.at[p],
            out_hbm.at[pl.ds(base + g * G, G), pl.ds(0, XCOLS), pl.ds(0, DIM)],
            osem[p])

    fire_group(0, 0)
    fire_group(1, 1)
    wait_group(0)
    write(0, 0).start()

    def substep(g, p):
        q = 1 - p
        write(g - 1, q).wait()
        fire_group(g + 1, q)
        wait_group(p)
        write(g, p).start()

    def body(i, carry):
        substep(2 * i + 1, 1)
        substep(2 * i + 2, 0)
        return carry

    lax.fori_loop(0, (NG - 2) // 2, body, 0)

    write(NG - 2, 0).wait()
    wait_group(1)
    write(NG - 1, 1).start()
    write(NG - 1, 1).wait()


def kernel(x, table):
    if x.dtype != jnp.int32:
        x = x.astype(jnp.int32)
    return _embed_gather(x, table)[:, :XCOLS, :DIM]


# speedup vs baseline: 1.0125x; 1.0125x over previous
import functools

import jax
import jax.numpy as jnp
from jax import lax
from jax.experimental import pallas as pl
from jax.experimental.pallas import tpu as pltpu
from jax.experimental.pallas import tpu_sc as plsc

NUM_ROWS = 100000
DIM = 64
XROWS = 4096
XCOLS = 50
NC = 2
NS = 16
NW = NC * NS
RPW = XROWS // NW
G = 8
NG = RPW // G

_mesh = plsc.VectorSubcoreMesh(core_axis_name="c", subcore_axis_name="s")


@functools.partial(
    pl.kernel,
    mesh=_mesh,
    compiler_params=pltpu.CompilerParams(use_tc_tiling_on_sc=False),
    out_type=jax.ShapeDtypeStruct((XROWS, 56, 128), jnp.float32),
    scratch_types=(
        [pltpu.VMEM((RPW, XCOLS), jnp.int32)]
        + [pltpu.VMEM((2, G, XCOLS, DIM), jnp.float32)]
        + [pltpu.SemaphoreType.DMA] * 4
    ),
)
def _embed_gather(x_hbm, table_hbm, out_hbm, idx_v, rows_v, g0, g1, o0, o1):
    gsem = (g0, g1)
    osem = (o0, o1)
    wid = lax.axis_index("s") * NC + lax.axis_index("c")
    base = wid * RPW
    pltpu.sync_copy(x_hbm.at[pl.ds(base, RPW)], idx_v)

    def fire_group(g, p):
        for k in range(G):
            pltpu.make_async_copy(
                table_hbm.at[idx_v.at[g * G + k]], rows_v.at[p].at[k],
                gsem[p]).start()

    def wait_group(p):
        for k in range(G):
            pltpu.make_async_copy(
                table_hbm.at[idx_v.at[0]], rows_v.at[p].at[k],
                gsem[p]).wait()

    def write(g, p):
        return pltpu.make_async_copy(
            rows_v.at[p],
            out_hbm.at[pl.ds(base + g * G, G), pl.ds(0, XCOLS), pl.ds(0, DIM)],
            osem[p])

    fire_group(0, 0)
    fire_group(1, 1)
    wait_group(0)
    write(0, 0).start()

    def substep(g, p):
        q = 1 - p
        write(g - 1, q).wait()
        fire_group(g + 1, q)
        wait_group(p)
        write(g, p).start()

    def body(i, carry):
        substep(2 * i + 1, 1)
        substep(2 * i + 2, 0)
        return carry

    lax.fori_loop(0, (NG - 2) // 2, body, 0)

    write(NG - 2, 0).wait()
    wait_group(1)
    write(NG - 1, 1).start()
    write(NG - 1, 1).wait()


def kernel(x, table):
    if x.dtype != jnp.int32:
        x = x.astype(jnp.int32)
    return _embed_gather(x, table)[:, :XCOLS, :DIM]
